# baseline (speedup 1.0000x reference)
import functools

import jax
import jax.numpy as jnp
from jax import lax
from jax.experimental import pallas as pl
from jax.experimental.pallas import tpu as pltpu
from jax.experimental.pallas import tpu_sc as plsc

N = 100000
D = 128
DIM_OUT = 40
BLOCK = 10000

_NC, _NS = 2, 16
_NW = _NC * _NS
_U = 2000
_NU = N // _U
_UPW = -(-_NU // _NW)


def _sc_prep_body(y_ref, m_ref, t_ref, yv, mv, tv):
    w = lax.axis_index("s") * _NC + lax.axis_index("c")
    for k in range(_UPW):
        u = w + k * _NW

        @pl.when(u < _NU)
        def _():
            base = u * _U
            pltpu.sync_copy(y_ref.at[pl.ds(base, _U)], yv)
            pltpu.sync_copy(m_ref.at[pl.ds(base, _U)], mv)
            for i in range(_U // 16):
                sl = pl.ds(i * 16, 16)
                tv[sl] = jnp.where(mv[sl] != 0, yv[sl], -1)
            pltpu.sync_copy(tv, t_ref.at[pl.ds(base, _U)])


_sc_prep = functools.partial(
    pl.kernel,
    out_type=jax.ShapeDtypeStruct((N,), jnp.int32),
    mesh=plsc.VectorSubcoreMesh(
        core_axis_name="c", subcore_axis_name="s",
        num_cores=_NC, num_subcores=_NS,
    ),
    scratch_types=[
        pltpu.VMEM((_U,), jnp.int32),
        pltpu.VMEM((_U,), jnp.int32),
        pltpu.VMEM((_U,), jnp.int32),
    ],
)(_sc_prep_body)


def _encode_block(x_ref, t_ref, o_ref):
    o_ref[:, :D] = x_ref[...]
    tcol = t_ref[0, 0, :].reshape(BLOCK, 1)
    cols = jax.lax.broadcasted_iota(jnp.int32, (BLOCK, DIM_OUT), 1)
    o_ref[:, D:] = (cols == tcol).astype(jnp.float32)


def kernel(x, y, train_mask):
    n = x.shape[0]
    grid = (n // BLOCK,)
    t = _sc_prep(y.reshape(n), train_mask.astype(jnp.int32))
    t = t.reshape(grid[0], 1, BLOCK)
    return pl.pallas_call(
        _encode_block,
        grid=grid,
        in_specs=[
            pl.BlockSpec((BLOCK, D), lambda i: (i, 0)),
            pl.BlockSpec((1, 1, BLOCK), lambda i: (i, 0, 0)),
        ],
        out_specs=pl.BlockSpec((BLOCK, D + DIM_OUT), lambda i: (i, 0)),
        out_shape=jax.ShapeDtypeStruct((n, D + DIM_OUT), x.dtype),
        compiler_params=pltpu.CompilerParams(
            dimension_semantics=("arbitrary",),
        ),
    )(x, t)


# speedup vs baseline: 1.1825x; 1.1825x over previous
import jax
import jax.numpy as jnp
from jax.experimental import pallas as pl
from jax.experimental.pallas import tpu as pltpu

N = 100000
D = 128
DIM_OUT = 40
CH = 5000
NCH = N // CH
NBUF = 4


def _pipe_body(x_hbm, t_ref, o_hbm, xbuf, obuf, insem, outsem):
    def in_copy(c, b):
        return pltpu.make_async_copy(
            x_hbm.at[pl.ds(c * CH, CH), :], xbuf.at[b], insem.at[b])

    def out_copy(c, b):
        return pltpu.make_async_copy(
            obuf.at[b], o_hbm.at[pl.ds(c * CH, CH), :], outsem.at[b])

    for c in range(NBUF):
        in_copy(c, c).start()

    cols = jax.lax.broadcasted_iota(jnp.int32, (CH, DIM_OUT), 1)
    for c in range(NCH):
        b = c % NBUF
        if c >= NBUF:
            out_copy(c - NBUF, b).wait()
        in_copy(c, b).wait()
        obuf[b, :, :D] = xbuf[b]
        if c + NBUF < NCH:
            in_copy(c + NBUF, b).start()
        tcol = t_ref[c, 0, :].reshape(CH, 1)
        obuf[b, :, D:] = (cols == tcol).astype(jnp.float32)
        out_copy(c, b).start()

    for c in range(NCH - NBUF, NCH):
        out_copy(c, c % NBUF).wait()


def kernel(x, y, train_mask):
    n = x.shape[0]
    t = jnp.where(train_mask, y[:, 0], -1).reshape(NCH, 1, CH)
    return pl.pallas_call(
        _pipe_body,
        in_specs=[
            pl.BlockSpec(memory_space=pltpu.MemorySpace.HBM),
            pl.BlockSpec(memory_space=pltpu.MemorySpace.VMEM),
        ],
        out_specs=pl.BlockSpec(memory_space=pltpu.MemorySpace.HBM),
        out_shape=jax.ShapeDtypeStruct((n, D + DIM_OUT), x.dtype),
        scratch_shapes=[
            pltpu.MemorySpace.VMEM((NBUF, CH, D), jnp.float32),
            pltpu.MemorySpace.VMEM((NBUF, CH, D + DIM_OUT), jnp.float32),
            pltpu.SemaphoreType.DMA((NBUF,)),
            pltpu.SemaphoreType.DMA((NBUF,)),
        ],
    )(x, t)
